# baseline (device time: 232556 ns/iter reference)
import jax
import jax.numpy as jnp
from jax import lax
from jax.experimental import pallas as pl
from jax.experimental.pallas import tpu as pltpu

N_DEV = 8
B_CH = 64
B_TOT = N_DEV * B_CH
D = 2048
H_LOC = 4096
K_T = 512
N_K = H_LOC // K_T
N_G = 2
G_ROWS = B_TOT // N_G
G_CH = N_DEV // N_G
MESH = pl.DeviceIdType.MESH


def _recv_wait(buf_ref, sem):
    pltpu.make_async_remote_copy(
        src_ref=buf_ref, dst_ref=buf_ref,
        send_sem=sem, recv_sem=sem,
        device_id=(0,), device_id_type=MESH,
    ).wait_recv()


def _layer(x_chunk, Win, Wout, cid):

    def body(x_ref, win_ref, wout_ref, out_ref,
             xg_ref, p_ref, psend_ref, prec_ref,
             ag_send, ag_recv, rs_send, rs_recv):
        g = pl.program_id(0)
        k = pl.program_id(1)
        my = lax.axis_index("i")

        @pl.when((g == 0) & (k == 0))
        def _start():
            bsem = pltpu.get_barrier_semaphore()
            for o in range(1, N_DEV):
                pl.semaphore_signal(
                    bsem, inc=1, device_id=((my + o) % N_DEV,),
                    device_id_type=MESH,
                )
            pl.semaphore_wait(bsem, N_DEV - 1)

            xg_ref[pl.ds(0, B_CH), :] = x_ref[...].astype(jnp.bfloat16)
            for o in range(1, N_DEV):
                q = N_DEV - o
                pltpu.make_async_remote_copy(
                    src_ref=xg_ref.at[pl.ds(0, B_CH), :],
                    dst_ref=xg_ref.at[pl.ds(q * B_CH, B_CH), :],
                    send_sem=ag_send.at[o],
                    recv_sem=ag_recv.at[q],
                    device_id=((my + o) % N_DEV,),
                    device_id_type=MESH,
                ).start()
            for q in range(1, G_CH):
                _recv_wait(xg_ref.at[pl.ds(q * B_CH, B_CH), :], ag_recv.at[q])

        @pl.when((g == 1) & (k == 0))
        def _wait_far_half():
            for q in range(G_CH, N_DEV):
                _recv_wait(xg_ref.at[pl.ds(q * B_CH, B_CH), :], ag_recv.at[q])

        xgg = xg_ref[pl.ds(g * G_ROWS, G_ROWS), :]
        wb = win_ref[...].astype(jnp.bfloat16)
        h = jnp.dot(xgg, wb, preferred_element_type=jnp.float32)
        hb = jnp.maximum(h, 0.0).astype(jnp.bfloat16)
        wo = wout_ref[...].astype(jnp.bfloat16)
        pp = jnp.dot(hb, wo, preferred_element_type=jnp.float32)

        @pl.when(k == 0)
        def _set():
            p_ref[pl.ds(g * G_ROWS, G_ROWS), :] = pp

        @pl.when(k > 0)
        def _acc():
            p_ref[pl.ds(g * G_ROWS, G_ROWS), :] += pp

        def _rs_group(gg):
            @pl.when((k == N_K - 1) & (g == gg))
            def _():
                psend_ref[pl.ds(gg * G_ROWS, G_ROWS), :] = (
                    p_ref[pl.ds(gg * G_ROWS, G_ROWS), :].astype(jnp.bfloat16))
                for q in range(gg * G_CH, (gg + 1) * G_CH):
                    if q == 0:
                        prec_ref[pl.ds(0, B_CH), :] = (
                            psend_ref[pl.ds(0, B_CH), :])
                        continue
                    pltpu.make_async_remote_copy(
                        src_ref=psend_ref.at[pl.ds(q * B_CH, B_CH), :],
                        dst_ref=prec_ref.at[pl.ds((N_DEV - q) * B_CH, B_CH), :],
                        send_sem=rs_send.at[q],
                        recv_sem=rs_recv.at[N_DEV - q],
                        device_id=((my + q) % N_DEV,),
                        device_id_type=MESH,
                    ).start()

        _rs_group(0)
        _rs_group(1)

        @pl.when((g == N_G - 1) & (k == N_K - 1))
        def _finish():
            for q in range(1, N_DEV):
                _recv_wait(prec_ref.at[pl.ds(q * B_CH, B_CH), :],
                           rs_recv.at[q])
            for o in range(1, N_DEV):
                pltpu.make_async_remote_copy(
                    src_ref=xg_ref.at[pl.ds(0, B_CH), :],
                    dst_ref=xg_ref.at[pl.ds(0, B_CH), :],
                    send_sem=ag_send.at[o], recv_sem=ag_recv.at[o],
                    device_id=(0,), device_id_type=MESH,
                ).wait_send()
            for q in range(1, N_DEV):
                pltpu.make_async_remote_copy(
                    src_ref=psend_ref.at[pl.ds(q * B_CH, B_CH), :],
                    dst_ref=psend_ref.at[pl.ds(q * B_CH, B_CH), :],
                    send_sem=rs_send.at[q], recv_sem=rs_recv.at[q],
                    device_id=(0,), device_id_type=MESH,
                ).wait_send()
            acc = prec_ref[pl.ds(0, B_CH), :].astype(jnp.float32)
            for q in range(1, N_DEV):
                acc = acc + prec_ref[pl.ds(q * B_CH, B_CH), :].astype(
                    jnp.float32)
            out_ref[...] = acc

    return pl.pallas_call(
        body,
        grid=(N_G, N_K),
        in_specs=[
            pl.BlockSpec((B_CH, D), lambda g, k: (0, 0)),
            pl.BlockSpec((D, K_T), lambda g, k: (0, k)),
            pl.BlockSpec((K_T, D), lambda g, k: (k, 0)),
        ],
        out_specs=pl.BlockSpec((B_CH, D), lambda g, k: (0, 0)),
        out_shape=jax.ShapeDtypeStruct((B_CH, D), jnp.float32),
        scratch_shapes=[
            pltpu.VMEM((B_TOT, D), jnp.bfloat16),
            pltpu.VMEM((B_TOT, D), jnp.float32),
            pltpu.VMEM((B_TOT, D), jnp.bfloat16),
            pltpu.VMEM((B_TOT, D), jnp.bfloat16),
            pltpu.SemaphoreType.DMA((N_DEV,)),
            pltpu.SemaphoreType.DMA((N_DEV,)),
            pltpu.SemaphoreType.DMA((N_DEV,)),
            pltpu.SemaphoreType.DMA((N_DEV,)),
        ],
        compiler_params=pltpu.CompilerParams(
            collective_id=cid,
            dimension_semantics=("arbitrary", "arbitrary"),
            vmem_limit_bytes=64 * 1024 * 1024,
        ),
    )(x_chunk, Win, Wout)


def kernel(x, Win0, Wout0, Win1, Wout1, Win2, Wout2):
    x = _layer(x, Win0, Wout0, 0)
    x = _layer(x, Win1, Wout1, 1)
    x = _layer(x, Win2, Wout2, 2)
    return x


# device time: 200221 ns/iter; 1.1615x vs baseline; 1.1615x over previous
import jax
import jax.numpy as jnp
from jax import lax
from jax.experimental import pallas as pl
from jax.experimental.pallas import tpu as pltpu

N_DEV = 8
B_CH = 64
D = 2048
H_LOC = 4096
K_T = 1024
N_K = H_LOC // K_T


def _layer(x_chunk, Win, Wout, cid):

    def body(x_ref, win_ref, wout_ref, out_ref,
             xg_ref, p_ref, psend_ref, prec_ref,
             ag_send, ag_recv, rs_send, rs_recv):
        k = pl.program_id(0)
        my = lax.axis_index("i")

        @pl.when(k == 0)
        def _allgather():
            bsem = pltpu.get_barrier_semaphore()
            for o in range(1, N_DEV):
                pl.semaphore_signal(
                    bsem, inc=1,
                    device_id=((my + o) % N_DEV,),
                    device_id_type=pl.DeviceIdType.MESH,
                )
            pl.semaphore_wait(bsem, N_DEV - 1)

            xg_ref[my] = x_ref[...].astype(jnp.bfloat16)

            sends = []
            for o in range(1, N_DEV):
                rdma = pltpu.make_async_remote_copy(
                    src_ref=xg_ref.at[my],
                    dst_ref=xg_ref.at[my],
                    send_sem=ag_send.at[o],
                    recv_sem=ag_recv.at[o],
                    device_id=((my + o) % N_DEV,),
                    device_id_type=pl.DeviceIdType.MESH,
                )
                rdma.start()
                sends.append(rdma)
            for o in range(1, N_DEV):
                pltpu.make_async_remote_copy(
                    src_ref=xg_ref.at[0],
                    dst_ref=xg_ref.at[0],
                    send_sem=ag_send.at[o],
                    recv_sem=ag_recv.at[o],
                    device_id=(0,),
                    device_id_type=pl.DeviceIdType.MESH,
                ).wait_recv()
            for rdma in sends:
                rdma.wait_send()
            p_ref[...] = jnp.zeros_like(p_ref)

        xg = xg_ref[...].reshape(N_DEV * B_CH, D)
        wb = win_ref[...].astype(jnp.bfloat16)
        h = jnp.dot(xg, wb, preferred_element_type=jnp.float32)
        hb = jnp.maximum(h, 0.0).astype(jnp.bfloat16)
        wo = wout_ref[...].astype(jnp.bfloat16)
        p_ref[...] += jnp.dot(hb, wo, preferred_element_type=jnp.float32)

        @pl.when(k == N_K - 1)
        def _reducescatter():
            psend_ref[...] = (
                p_ref[...].reshape(N_DEV, B_CH, D).astype(jnp.bfloat16)
            )
            prec_ref[my] = psend_ref[my]
            sends = []
            for o in range(1, N_DEV):
                rdma = pltpu.make_async_remote_copy(
                    src_ref=psend_ref.at[(my + o) % N_DEV],
                    dst_ref=prec_ref.at[my],
                    send_sem=rs_send.at[o],
                    recv_sem=rs_recv.at[o],
                    device_id=((my + o) % N_DEV,),
                    device_id_type=pl.DeviceIdType.MESH,
                )
                rdma.start()
                sends.append(rdma)
            for o in range(1, N_DEV):
                pltpu.make_async_remote_copy(
                    src_ref=prec_ref.at[0],
                    dst_ref=prec_ref.at[0],
                    send_sem=rs_send.at[o],
                    recv_sem=rs_recv.at[o],
                    device_id=(0,),
                    device_id_type=pl.DeviceIdType.MESH,
                ).wait_recv()
            for rdma in sends:
                rdma.wait_send()
            acc = prec_ref[0].astype(jnp.float32)
            for j in range(1, N_DEV):
                acc = acc + prec_ref[j].astype(jnp.float32)
            out_ref[...] = acc

    return pl.pallas_call(
        body,
        grid=(N_K,),
        in_specs=[
            pl.BlockSpec((B_CH, D), lambda k: (0, 0)),
            pl.BlockSpec((D, K_T), lambda k: (0, k)),
            pl.BlockSpec((K_T, D), lambda k: (k, 0)),
        ],
        out_specs=pl.BlockSpec((B_CH, D), lambda k: (0, 0)),
        out_shape=jax.ShapeDtypeStruct((B_CH, D), jnp.float32),
        scratch_shapes=[
            pltpu.VMEM((N_DEV, B_CH, D), jnp.bfloat16),
            pltpu.VMEM((N_DEV * B_CH, D), jnp.float32),
            pltpu.VMEM((N_DEV, B_CH, D), jnp.bfloat16),
            pltpu.VMEM((N_DEV, B_CH, D), jnp.bfloat16),
            pltpu.SemaphoreType.DMA((N_DEV,)),
            pltpu.SemaphoreType.DMA((N_DEV,)),
            pltpu.SemaphoreType.DMA((N_DEV,)),
            pltpu.SemaphoreType.DMA((N_DEV,)),
        ],
        compiler_params=pltpu.CompilerParams(
            collective_id=cid,
            dimension_semantics=("arbitrary",),
            vmem_limit_bytes=64 * 1024 * 1024,
        ),
    )(x_chunk, Win, Wout)


def kernel(x, Win0, Wout0, Win1, Wout1, Win2, Wout2):
    x = _layer(x, Win0, Wout0, 0)
    x = _layer(x, Win1, Wout1, 1)
    x = _layer(x, Win2, Wout2, 2)
    return x


# device time: 185099 ns/iter; 1.2564x vs baseline; 1.0817x over previous
import jax
import jax.numpy as jnp
from jax import lax
from jax.experimental import pallas as pl
from jax.experimental.pallas import tpu as pltpu

N_DEV = 8
N_L = 3
B_CH = 64
D = 2048
H_LOC = 4096
K_T = 512
N_K = H_LOC // K_T
MESH = pl.DeviceIdType.MESH


def kernel(x, Win0, Wout0, Win1, Wout1, Win2, Wout2):
    def body(x_ref, w0_ref, wo0_ref, w1_ref, wo1_ref, w2_ref, wo2_ref,
             out_ref,
             xc_ref, xg_ref, p_ref, psend_ref, prec_ref,
             ag_send, ag_recv, rs_send, rs_recv):
        l = pl.program_id(0)
        k = pl.program_id(1)
        my = lax.axis_index("i")

        @pl.when((l == 0) & (k == 0))
        def _barrier():
            bsem = pltpu.get_barrier_semaphore()
            for o in range(1, N_DEV):
                pl.semaphore_signal(
                    bsem, inc=1, device_id=((my + o) % N_DEV,),
                    device_id_type=MESH,
                )
            pl.semaphore_wait(bsem, N_DEV - 1)
            xc_ref[...] = x_ref[...]

        @pl.when(k == 0)
        def _allgather():
            xg_ref[my] = xc_ref[...].astype(jnp.bfloat16)
            sends = []
            for o in range(1, N_DEV):
                rdma = pltpu.make_async_remote_copy(
                    src_ref=xg_ref.at[my],
                    dst_ref=xg_ref.at[my],
                    send_sem=ag_send.at[o],
                    recv_sem=ag_recv.at[o],
                    device_id=((my + o) % N_DEV,),
                    device_id_type=MESH,
                )
                rdma.start()
                sends.append(rdma)
            for o in range(1, N_DEV):
                pltpu.make_async_remote_copy(
                    src_ref=xg_ref.at[0], dst_ref=xg_ref.at[0],
                    send_sem=ag_send.at[o], recv_sem=ag_recv.at[o],
                    device_id=(0,), device_id_type=MESH,
                ).wait_recv()
            for rdma in sends:
                rdma.wait_send()

        def compute(i, win_ref, wout_ref):
            @pl.when(l == i)
            def _():
                xg = xg_ref[...].reshape(N_DEV * B_CH, D)
                wb = win_ref[...].astype(jnp.bfloat16)
                h = jnp.dot(xg, wb, preferred_element_type=jnp.float32)
                hb = jnp.maximum(h, 0.0).astype(jnp.bfloat16)
                wo = wout_ref[...].astype(jnp.bfloat16)
                pp = jnp.dot(hb, wo, preferred_element_type=jnp.float32)

                @pl.when(k == 0)
                def _set():
                    p_ref[...] = pp

                @pl.when((k > 0) & (k < N_K - 1))
                def _acc():
                    p_ref[...] += pp

                @pl.when(k == N_K - 1)
                def _fin():
                    pfin = p_ref[...] + pp
                    psend_ref[...] = pfin.reshape(
                        N_DEV, B_CH, D).astype(jnp.bfloat16)

        compute(0, w0_ref, wo0_ref)
        compute(1, w1_ref, wo1_ref)
        compute(2, w2_ref, wo2_ref)

        @pl.when(k == N_K - 1)
        def _reducescatter():
            prec_ref[my] = psend_ref[my]
            sends = []
            for o in range(1, N_DEV):
                rdma = pltpu.make_async_remote_copy(
                    src_ref=psend_ref.at[(my + o) % N_DEV],
                    dst_ref=prec_ref.at[my],
                    send_sem=rs_send.at[o],
                    recv_sem=rs_recv.at[o],
                    device_id=((my + o) % N_DEV,),
                    device_id_type=MESH,
                )
                rdma.start()
                sends.append(rdma)
            for o in range(1, N_DEV):
                pltpu.make_async_remote_copy(
                    src_ref=prec_ref.at[0], dst_ref=prec_ref.at[0],
                    send_sem=rs_send.at[o], recv_sem=rs_recv.at[o],
                    device_id=(0,), device_id_type=MESH,
                ).wait_recv()
            for rdma in sends:
                rdma.wait_send()
            acc = prec_ref[0].astype(jnp.float32)
            for j in range(1, N_DEV):
                acc = acc + prec_ref[j].astype(jnp.float32)
            xc_ref[...] = acc

        @pl.when((l == N_L - 1) & (k == N_K - 1))
        def _out():
            out_ref[...] = xc_ref[...]

    def win_map(i):
        return lambda l, k: (
            0, jnp.where(l < i, 0, jnp.where(l > i, N_K - 1, k)))

    def wout_map(i):
        return lambda l, k: (
            jnp.where(l < i, 0, jnp.where(l > i, N_K - 1, k)), 0)

    return pl.pallas_call(
        body,
        grid=(N_L, N_K),
        in_specs=[
            pl.BlockSpec((B_CH, D), lambda l, k: (0, 0)),
            pl.BlockSpec((D, K_T), win_map(0)),
            pl.BlockSpec((K_T, D), wout_map(0)),
            pl.BlockSpec((D, K_T), win_map(1)),
            pl.BlockSpec((K_T, D), wout_map(1)),
            pl.BlockSpec((D, K_T), win_map(2)),
            pl.BlockSpec((K_T, D), wout_map(2)),
        ],
        out_specs=pl.BlockSpec((B_CH, D), lambda l, k: (0, 0)),
        out_shape=jax.ShapeDtypeStruct((B_CH, D), jnp.float32),
        scratch_shapes=[
            pltpu.VMEM((B_CH, D), jnp.float32),
            pltpu.VMEM((N_DEV, B_CH, D), jnp.bfloat16),
            pltpu.VMEM((N_DEV * B_CH, D), jnp.float32),
            pltpu.VMEM((N_DEV, B_CH, D), jnp.bfloat16),
            pltpu.VMEM((N_DEV, B_CH, D), jnp.bfloat16),
            pltpu.SemaphoreType.DMA((N_DEV,)),
            pltpu.SemaphoreType.DMA((N_DEV,)),
            pltpu.SemaphoreType.DMA((N_DEV,)),
            pltpu.SemaphoreType.DMA((N_DEV,)),
        ],
        compiler_params=pltpu.CompilerParams(
            collective_id=0,
            dimension_semantics=("arbitrary", "arbitrary"),
            vmem_limit_bytes=64 * 1024 * 1024,
        ),
    )(x, Win0, Wout0, Win1, Wout1, Win2, Wout2)
